# baseline (speedup 1.0000x reference)
import jax
import jax.numpy as jnp
from jax import lax
from jax.experimental import pallas as pl
from jax.experimental.pallas import tpu as pltpu
from jax.experimental.pallas import tpu_sc as plsc

_NC = 2
_NS = 16
_NW = _NC * _NS
_LANES = 16

_ROWS = 50000
_COLS = 256
_NIDX = 20000
_TRIG = 64

_CHUNKS = 5
_CHUNK = 128
_PER_W = _CHUNKS * _CHUNK
_NIDX_PAD = _NW * _PER_W

_TC_SEG = 1568


def _tc_copy_body(x_ref, o_ref):
    o_ref[...] = x_ref[...]


def _tc_copy(x):
    grid = (_ROWS + _TC_SEG - 1) // _TC_SEG
    return pl.pallas_call(
        _tc_copy_body,
        grid=(grid,),
        in_specs=[pl.BlockSpec((_TC_SEG, _COLS), lambda i: (i, 0))],
        out_specs=pl.BlockSpec((_TC_SEG, _COLS), lambda i: (i, 0)),
        out_shape=jax.ShapeDtypeStruct((_ROWS, _COLS), jnp.float32),
    )(x)


def _sc_update_body(x2_hbm, able2d_hbm, trig_hbm, out2_ref, idx2d, rows_v, trig_v, sem):
    wid = lax.axis_index("s") * _NC + lax.axis_index("c")

    pltpu.sync_copy(able2d_hbm.at[wid], idx2d)
    pltpu.sync_copy(trig_hbm, trig_v)

    for j in range(_CHUNKS):
        for k in range(_CHUNK // _LANES):
            sl = pl.ds(k * _LANES, _LANES)
            idx2d[j, sl] = idx2d[j, sl] * 4

    gathers = [
        pltpu.async_copy(
            x2_hbm.at[idx2d.at[j]], rows_v.at[pl.ds(j * _CHUNK, _CHUNK)], sem
        )
        for j in range(_CHUNKS)
    ]
    for g in gathers:
        g.wait()

    tvecs = [trig_v[pl.ds(c * _LANES, _LANES)] for c in range(_TRIG // _LANES)]

    def row_body(r, _):
        for c in range(_TRIG // _LANES):
            sl = pl.ds(c * _LANES, _LANES)
            rows_v[r, sl] = jnp.minimum(rows_v[r, sl] + tvecs[c], 1.0)
        return _

    lax.fori_loop(0, _PER_W, row_body, None)

    scatters = [
        pltpu.async_copy(
            rows_v.at[pl.ds(j * _CHUNK, _CHUNK)], out2_ref.at[idx2d.at[j]], sem
        )
        for j in range(_CHUNKS)
    ]
    for s in scatters:
        s.wait()


def _sc_update(x2, able2d, trigger, out_ref):
    mesh = plsc.VectorSubcoreMesh(core_axis_name="c", subcore_axis_name="s")
    pl.kernel(
        _sc_update_body,
        out_type=(),
        mesh=mesh,
        scratch_types=[
            pltpu.VMEM((_CHUNKS, _CHUNK), jnp.int32),
            pltpu.VMEM((_PER_W, _TRIG), jnp.float32),
            pltpu.VMEM((_TRIG,), jnp.float32),
            pltpu.SemaphoreType.DMA,
        ],
        compiler_params=pltpu.CompilerParams(
            needs_layout_passes=False, use_tc_tiling_on_sc=False
        ),
    )(x2, able2d, trigger, out_ref)


def kernel(x, able, trigger):
    able = able.astype(jnp.int32)
    able_p = jnp.concatenate(
        [able, jnp.broadcast_to(able[:1], (_NIDX_PAD - _NIDX,))]
    )
    able2d = able_p.reshape(_NW, _CHUNKS, _CHUNK)
    x2 = x.reshape(_ROWS * 4, _TRIG)

    out = _tc_copy(x)
    out_ref = jax.new_ref(out.reshape(_ROWS * 4, _TRIG))
    _sc_update(x2, able2d, trigger.astype(jnp.float32), out_ref)
    return jax.freeze(out_ref).reshape(_ROWS, _COLS)
